# baseline (speedup 1.0000x reference)
import functools

import jax
import jax.numpy as jnp
from jax import lax
from jax.experimental import pallas as pl
from jax.experimental.pallas import tpu as pltpu
from jax.experimental.pallas import tpu_sc as plsc

NC = 2
NS = 16
L = 16
NW = NC * NS

B = 16384
D = 16
BPW = B // NW
NCHUNK = 4
CHUNK = BPW // NCHUNK

RPP = 128 // D
BIAS_PAGES = 7813


def _svdpp_body(x_hbm, ue_hbm, ie_hbm, ub_hbm, ib_hbm, out_hbm,
                x_v, upg_v, ipg_v, ubp_v, ibp_v, uoff_v, ioff_v,
                ubo_v, ibo_v, upage_v, ipage_v, ubpage_v, ibpage_v, out_v,
                sem):
    wid = lax.axis_index("s") * NC + lax.axis_index("c")
    base = wid * BPW

    pltpu.sync_copy(x_hbm.at[pl.ds(base * 2, BPW * 2)], x_v)

    iota = lax.iota(jnp.int32, L)

    for j in range(NCHUNK):
        for i in range(CHUNK // L):
            r = j * CHUNK + i * L
            flat = (iota + r) * 2
            u = plsc.load_gather(x_v, [flat])
            v = plsc.load_gather(x_v, [flat + 1])
            upg_v[j, pl.ds(i * L, L)] = u >> 3
            ipg_v[j, pl.ds(i * L, L)] = v >> 3
            ubp_v[j, pl.ds(i * L, L)] = u >> 7
            ibp_v[j, pl.ds(i * L, L)] = v >> 7
            uoff_v[pl.ds(r, L)] = (u & 7) << 4
            ioff_v[pl.ds(r, L)] = (v & 7) << 4
            ubo_v[pl.ds(r, L)] = u & 127
            ibo_v[pl.ds(r, L)] = v & 127

    for j in range(NCHUNK):
        cu = pltpu.async_copy(ue_hbm.at[upg_v.at[j]], upage_v, sem)
        ci = pltpu.async_copy(ie_hbm.at[ipg_v.at[j]], ipage_v, sem)
        cub = pltpu.async_copy(ub_hbm.at[ubp_v.at[j]], ubpage_v, sem)
        cib = pltpu.async_copy(ib_hbm.at[ibp_v.at[j]], ibpage_v, sem)
        cu.wait()
        ci.wait()
        cub.wait()
        cib.wait()

        for t in range(CHUNK // L):
            r = j * CHUNK + t * L
            rows = iota + t * L
            uo = uoff_v[pl.ds(r, L)]
            io = ioff_v[pl.ds(r, L)]
            acc = jnp.zeros((L,), jnp.float32)
            s = jnp.zeros((L,), jnp.float32)
            for c in range(D):
                u = plsc.load_gather(upage_v, [rows, uo + c])
                v = plsc.load_gather(ipage_v, [rows, io + c])
                acc = acc + u * v
                s = s + u
            ub = plsc.load_gather(ubpage_v, [rows, ubo_v[pl.ds(r, L)]])
            ib = plsc.load_gather(ibpage_v, [rows, ibo_v[pl.ds(r, L)]])
            z = acc + ub + ib + s * (1.0 / D)
            out_v[pl.ds(r, L)] = 1.0 / (1.0 + jnp.exp(-z))

    pltpu.sync_copy(out_v, out_hbm.at[pl.ds(base, BPW)])


@jax.jit
def kernel(x, user_emb, item_emb, user_bias, item_bias):
    xf = x.reshape(-1)
    uep = user_emb.reshape(-1, 128)
    iep = item_emb.reshape(-1, 128)
    npad = BIAS_PAGES * 128 - user_bias.shape[0]
    ubp = jnp.pad(user_bias.reshape(-1), (0, npad)).reshape(BIAS_PAGES, 128)
    ibp = jnp.pad(item_bias.reshape(-1), (0, npad)).reshape(BIAS_PAGES, 128)
    mesh = plsc.VectorSubcoreMesh(core_axis_name="c", subcore_axis_name="s",
                                  num_cores=NC, num_subcores=NS)
    run = pl.kernel(
        _svdpp_body,
        out_type=jax.ShapeDtypeStruct((B,), jnp.float32),
        mesh=mesh,
        compiler_params=pltpu.CompilerParams(needs_layout_passes=False),
        scratch_types=[
            pltpu.VMEM((BPW * 2,), jnp.int32),
            pltpu.VMEM((NCHUNK, CHUNK), jnp.int32),
            pltpu.VMEM((NCHUNK, CHUNK), jnp.int32),
            pltpu.VMEM((NCHUNK, CHUNK), jnp.int32),
            pltpu.VMEM((NCHUNK, CHUNK), jnp.int32),
            pltpu.VMEM((BPW,), jnp.int32),
            pltpu.VMEM((BPW,), jnp.int32),
            pltpu.VMEM((BPW,), jnp.int32),
            pltpu.VMEM((BPW,), jnp.int32),
            pltpu.VMEM((CHUNK, 128), jnp.float32),
            pltpu.VMEM((CHUNK, 128), jnp.float32),
            pltpu.VMEM((CHUNK, 128), jnp.float32),
            pltpu.VMEM((CHUNK, 128), jnp.float32),
            pltpu.VMEM((BPW,), jnp.float32),
            pltpu.SemaphoreType.DMA,
        ],
    )
    return run(xf, uep, iep, ubp, ibp)
